# baseline (speedup 1.0000x reference)
import functools

import numpy as np

import jax
import jax.numpy as jnp
from jax import lax
from jax.experimental import pallas as pl
from jax.experimental.pallas import tpu as pltpu
from jax.experimental.pallas import tpu_sc as plsc

NP_ = 8000
NL_ = 2000
N_ = NP_ + NL_
NPAD = 10240
FP = 27
FL = 13
H = 128
B = 64
K = 32
NLAYERS = 3
NG = 20
NGP = 24
CUTOFF = 10.0
OUT = 3

RB = 128
CT = 128
BD = 128
BIGM = 1e10
INF = 3e30

_NTC = NPAD // CT
_NRB = NPAD // RB
_NBD = NPAD // BD
TOT = K * NPAD


def _embed_body(f_ref, w_ref, b_ref, o_ref):
    o_ref[...] = jnp.dot(f_ref[...], w_ref[...],
                         preferred_element_type=jnp.float32) + b_ref[...]


def _embed(feat, w, b, blk):
    n, f = feat.shape
    return pl.pallas_call(
        _embed_body,
        grid=(n // blk,),
        in_specs=[
            pl.BlockSpec((blk, f), lambda i: (i, 0)),
            pl.BlockSpec((f, H), lambda i: (0, 0)),
            pl.BlockSpec((1, H), lambda i: (0, 0)),
        ],
        out_specs=pl.BlockSpec((blk, H), lambda i: (i, 0)),
        out_shape=jax.ShapeDtypeStruct((n, H), jnp.float32),
    )(feat, w, b.reshape(1, H))


def _sc_permute(h_un, pq_un, idx):
    mesh = plsc.VectorSubcoreMesh(core_axis_name="c", subcore_axis_name="s")
    info = plsc.get_sparse_core_info()
    nc = info.num_cores

    @functools.partial(
        pl.kernel, mesh=mesh,
        out_type=[jax.ShapeDtypeStruct((NPAD, H), jnp.float32),
                  jax.ShapeDtypeStruct((NPAD, H), jnp.float32)],
        scratch_types=[
            pltpu.VMEM((320,), jnp.int32),
            pltpu.VMEM((64, H), jnp.float32),
            pltpu.VMEM((64, H), jnp.float32),
            pltpu.SemaphoreType.DMA,
        ],
    )
    def k(h_hbm, pq_hbm, idx_hbm, oh_hbm, opq_hbm, idx_v, hrows, pqrows, sem):
        wid = lax.axis_index("s") * nc + lax.axis_index("c")
        pltpu.sync_copy(idx_hbm.at[pl.ds(wid * 320, 320)], idx_v)

        def body(j, _):
            base = wid * 320 + j * 64
            ids = idx_v.at[pl.ds(j * 64, 64)]
            pltpu.async_copy(h_hbm.at[ids], hrows, sem).wait()
            pltpu.sync_copy(hrows, oh_hbm.at[pl.ds(base, 64)])
            pltpu.async_copy(pq_hbm.at[ids], pqrows, sem).wait()
            pltpu.sync_copy(pqrows, opq_hbm.at[pl.ds(base, 64)])
            return 0

        lax.fori_loop(0, 5, body, 0)

    return k(h_un, pq_un, idx)


def _sc_gather(table, idx):
    mesh = plsc.VectorSubcoreMesh(core_axis_name="c", subcore_axis_name="s")
    info = plsc.get_sparse_core_info()
    nc = info.num_cores

    @functools.partial(
        pl.kernel, mesh=mesh,
        out_type=jax.ShapeDtypeStruct((TOT, H), jnp.float32),
        scratch_types=[
            pltpu.VMEM((10240,), jnp.int32),
            pltpu.VMEM((128, H), jnp.float32),
            pltpu.VMEM((128, H), jnp.float32),
            pltpu.VMEM((128, H), jnp.float32),
            pltpu.VMEM((128, H), jnp.float32),
            pltpu.SemaphoreType.DMA,
            pltpu.SemaphoreType.DMA,
            pltpu.SemaphoreType.DMA,
            pltpu.SemaphoreType.DMA,
            pltpu.SemaphoreType.DMA,
            pltpu.SemaphoreType.DMA,
            pltpu.SemaphoreType.DMA,
            pltpu.SemaphoreType.DMA,
        ],
    )
    def k(tbl_hbm, idx_hbm, out_hbm, idx_v, r0, r1, r2, r3,
          g0, g1, g2, g3, w0, w1, w2, w3):
        wid = lax.axis_index("s") * nc + lax.axis_index("c")
        base = wid * 10240
        pltpu.sync_copy(idx_hbm.at[pl.ds(base, 10240)], idx_v)
        rows = (r0, r1, r2, r3)
        gsem = (g0, g1, g2, g3)
        wsem = (w0, w1, w2, w3)
        for b in range(4):
            pltpu.async_copy(
                tbl_hbm.at[idx_v.at[pl.ds(b * 128, 128)]], rows[b], gsem[b])

        def body(j4, _):
            for b in range(4):
                j = 4 * j4 + b
                pltpu.make_async_copy(
                    tbl_hbm.at[pl.ds(0, 128)], rows[b], gsem[b]).wait()
                pltpu.async_copy(
                    rows[b], out_hbm.at[pl.ds(base + j * 128, 128)], wsem[b])

                @pl.when(j + 4 < 80)
                def _():
                    pltpu.make_async_copy(
                        rows[b], out_hbm.at[pl.ds(0, 128)], wsem[b]).wait()
                    ids = idx_v.at[pl.ds((j + 4) * 128, 128)]
                    pltpu.async_copy(tbl_hbm.at[ids], rows[b], gsem[b])
            return 0

        lax.fori_loop(0, 20, body, 0)
        for b in range(4):
            pltpu.make_async_copy(
                rows[b], out_hbm.at[pl.ds(0, 128)], wsem[b]).wait()

    return k(table, idx)


def _knn_body(jlo_ref, nt_ref, prow_ref, pq_any, outd_ref, outi_ref,
              ptile, sem):
    i = pl.program_id(0)
    pr = prow_ref[...]
    lane16 = lax.broadcasted_iota(jnp.int32, (RB, 16), 1)
    xyz_m = (lane16 < 3).astype(jnp.float32)
    pxyz_r = pr * xyz_m
    sq_r = jnp.sum(pxyz_r * pxyz_r, axis=1, keepdims=True)
    b_r = pr[:, 3:4]
    ar = pxyz_r + jnp.where(lane16 == 3, sq_r, 0.0) \
        + jnp.where(lane16 == 4, 1.0, 0.0)
    rg = (i * RB + lax.broadcasted_iota(jnp.int32, (RB, 1), 0)).astype(
        jnp.float32)

    jlo = jlo_ref[i]
    nt = nt_ref[i]

    pltpu.make_async_copy(pq_any.at[pl.ds(jlo * CT, CT)], ptile.at[0],
                          sem.at[0]).start()

    def tile_step(t, carry):
        bd, bi = carry
        jt = jlo + t
        p = lax.rem(t, 2)
        pltpu.make_async_copy(pq_any.at[pl.ds(jt * CT, CT)], ptile.at[p],
                              sem.at[p]).wait()

        @pl.when(t + 1 < nt)
        def _():
            pltpu.make_async_copy(
                pq_any.at[pl.ds((jt + 1) * CT, CT)],
                ptile.at[lax.rem(t + 1, 2)], sem.at[lax.rem(t + 1, 2)]).start()

        pc = ptile[p]
        lane16c = lax.broadcasted_iota(jnp.int32, (CT, 16), 1)
        pxyz_c = pc * (lane16c < 3).astype(jnp.float32)
        sq_c = jnp.sum(pxyz_c * pxyz_c, axis=1, keepdims=True)
        b_c = pc[:, 3:4]
        ac = -2.0 * pxyz_c + jnp.where(lane16c == 3, 1.0, 0.0) \
            + jnp.where(lane16c == 4, sq_c, 0.0)
        d2 = lax.dot_general(ar, ac, (((1,), (1,)), ((), ())),
                             preferred_element_type=jnp.float32)
        bm = lax.dot_general(
            jnp.concatenate([b_r * b_r, b_r, jnp.ones_like(b_r)], axis=1),
            jnp.concatenate([jnp.ones_like(b_c), -2.0 * b_c, b_c * b_c],
                            axis=1),
            (((1,), (1,)), ((), ())),
            preferred_element_type=jnp.float32)
        cg = (jt * CT
              + lax.broadcasted_iota(jnp.int32, (RB, CT), 1)).astype(
                  jnp.float32)
        d2 = jnp.where(bm != 0.0, BIGM, d2)
        d2 = jnp.where(cg == rg, BIGM, d2)
        lanes = lax.broadcasted_iota(jnp.int32, (RB, K), 1)
        nbd = jnp.full((RB, K), INF, jnp.float32)
        nbi = jnp.zeros((RB, K), jnp.float32)
        for tt in range(K):
            m = jnp.minimum(jnp.min(bd, axis=1, keepdims=True),
                            jnp.min(d2, axis=1, keepdims=True))
            za = jnp.where(bd == m, bi, 3e30)
            zb = jnp.where(d2 == m, cg, 3e30)
            gi = jnp.minimum(jnp.min(za, axis=1, keepdims=True),
                             jnp.min(zb, axis=1, keepdims=True))
            bd = jnp.where(za == gi, INF, bd)
            d2 = jnp.where(zb == gi, INF, d2)
            sel = lanes == tt
            nbd = jnp.where(sel, m, nbd)
            nbi = jnp.where(sel, gi, nbi)
        return (nbd, nbi)

    bd0 = jnp.full((RB, K), INF, jnp.float32)
    bi0 = jnp.zeros((RB, K), jnp.float32)
    bd, bi = lax.fori_loop(0, nt, tile_step, (bd0, bi0))
    outd_ref[...] = bd
    outi_ref[...] = jnp.transpose(bi.astype(jnp.int32))


def _knn(pq, jlo, ntiles):
    grid_spec = pltpu.PrefetchScalarGridSpec(
        num_scalar_prefetch=2,
        grid=(_NRB,),
        in_specs=[
            pl.BlockSpec((RB, 16), lambda i, jlo, nt: (i, 0)),
            pl.BlockSpec(memory_space=pl.ANY),
        ],
        out_specs=[
            pl.BlockSpec((RB, K), lambda i, jlo, nt: (i, 0)),
            pl.BlockSpec((K, RB), lambda i, jlo, nt: (0, i)),
        ],
        scratch_shapes=[
            pltpu.VMEM((2, CT, 16), jnp.float32),
            pltpu.SemaphoreType.DMA((2,)),
        ],
    )
    return pl.pallas_call(
        _knn_body,
        grid_spec=grid_spec,
        out_shape=[jax.ShapeDtypeStruct((NPAD, K), jnp.float32),
                   jax.ShapeDtypeStruct((K, NPAD), jnp.int32)],
    )(jlo, ntiles, pq, pq)


_CSTEP = float(CUTOFF) / (NG - 1)
_W = CUTOFF / NG
_INV2W2 = 1.0 / (2.0 * _W * _W)


def _layer_body(h_ref, bs_ref, d2_ref, w1d_ref, w1e_ref, b1_ref, w2_ref,
                b2_ref, wn1h_ref, wn1a_ref, bn1_ref, wn2_ref, bn2_ref,
                g_ref, be_ref, wnext_ref, oh_ref, ob_ref, *, emit_next):
    h = h_ref[...]
    a = jnp.dot(h, w1d_ref[...], preferred_element_type=jnp.float32) \
        + b1_ref[...]
    cen = lax.broadcasted_iota(jnp.int32, (1, NGP), 1).astype(
        jnp.float32) * _CSTEP
    s0 = jnp.zeros((BD, H), jnp.float32)
    s1 = jnp.zeros((BD, H), jnp.float32)
    for k in range(K):
        dk = d2_ref[:, k:k + 1]
        dist = jnp.sqrt(dk + 1e-12)
        ek = jnp.exp(-((dist - cen) ** 2) * _INV2W2)
        m1 = a + bs_ref[k] + jnp.dot(ek, w1e_ref[...],
                                     preferred_element_type=jnp.float32)
        if k % 2 == 0:
            s0 = s0 + jnp.maximum(m1, 0.0)
        else:
            s1 = s1 + jnp.maximum(m1, 0.0)
    agg = jnp.dot(s0 + s1, w2_ref[...], preferred_element_type=jnp.float32) \
        + float(K) * b2_ref[...]
    u = jnp.maximum(
        jnp.dot(h, wn1h_ref[...], preferred_element_type=jnp.float32)
        + jnp.dot(agg, wn1a_ref[...], preferred_element_type=jnp.float32)
        + bn1_ref[...], 0.0)
    u = jnp.dot(u, wn2_ref[...], preferred_element_type=jnp.float32) \
        + bn2_ref[...]
    r = h + u
    mu = jnp.mean(r, axis=1, keepdims=True)
    var = jnp.mean((r - mu) ** 2, axis=1, keepdims=True)
    hn = (r - mu) * lax.rsqrt(var + 1e-5) * g_ref[...] + be_ref[...]
    oh_ref[...] = hn
    if emit_next:
        ob_ref[...] = jnp.dot(hn, wnext_ref[...],
                              preferred_element_type=jnp.float32)


def _layer(h, bs, d2k, wd, emit_next):
    row = lambda i: (i, 0)
    fix = lambda i: (0, 0)
    in_specs = [
        pl.BlockSpec((BD, H), row),
        pl.BlockSpec((K, BD, H), lambda i: (0, i, 0)),
        pl.BlockSpec((BD, K), row),
        pl.BlockSpec((H, H), fix),
        pl.BlockSpec((NGP, H), fix),
        pl.BlockSpec((1, H), fix),
        pl.BlockSpec((H, H), fix),
        pl.BlockSpec((1, H), fix),
        pl.BlockSpec((H, H), fix),
        pl.BlockSpec((H, H), fix),
        pl.BlockSpec((1, H), fix),
        pl.BlockSpec((H, H), fix),
        pl.BlockSpec((1, H), fix),
        pl.BlockSpec((1, H), fix),
        pl.BlockSpec((1, H), fix),
        pl.BlockSpec((H, H), fix),
    ]
    out_shape = [jax.ShapeDtypeStruct((NPAD, H), jnp.float32),
                 jax.ShapeDtypeStruct((NPAD, H), jnp.float32)]
    out_specs = [pl.BlockSpec((BD, H), row), pl.BlockSpec((BD, H), row)]
    outs = pl.pallas_call(
        functools.partial(_layer_body, emit_next=emit_next),
        grid=(_NBD,),
        in_specs=in_specs,
        out_specs=out_specs,
        out_shape=out_shape,
    )(h, bs, d2k, *wd)
    return outs


def _readout_body(h_ref, pq_ref, kind_ref, w1_ref, b1_ref, w2_ref, b2_ref,
                  out_ref, acc):
    i = pl.program_id(0)

    @pl.when(i == 0)
    def _():
        acc[...] = jnp.zeros((B, H), jnp.float32)

    batch_r = pq_ref[:, 3:4]
    bl = lax.broadcasted_iota(jnp.int32, (RB, B), 1).astype(jnp.float32)
    e = (batch_r == bl).astype(jnp.float32)
    acc[...] += lax.dot_general(e, h_ref[...], (((0,), (0,)), ((), ())),
                                preferred_element_type=jnp.float32)

    @pl.when(i == _NRB - 1)
    def _():
        pre = acc[...]
        x = jnp.dot(pre, w1_ref[...], preferred_element_type=jnp.float32) \
            + b1_ref[...]
        sp = jnp.maximum(x, 0.0) + jnp.log1p(jnp.exp(-jnp.abs(x))) \
            - np.float32(np.log(2.0))
        o3 = jnp.dot(sp, w2_ref[...], preferred_element_type=jnp.float32) \
            + b2_ref[...]
        sel = (lax.broadcasted_iota(jnp.int32, (B, OUT), 1).astype(
            jnp.float32) == kind_ref[...] - 1.0).astype(jnp.float32)
        out_ref[...] = jnp.sum(o3 * sel, axis=1, keepdims=True)


def _readout(h, pq, kindf, w1, b1, w2, b2):
    fix = lambda i: (0, 0)
    return pl.pallas_call(
        _readout_body,
        grid=(_NRB,),
        in_specs=[
            pl.BlockSpec((RB, H), lambda i: (i, 0)),
            pl.BlockSpec((RB, 16), lambda i: (i, 0)),
            pl.BlockSpec((B, 1), fix),
            pl.BlockSpec((H, H), fix),
            pl.BlockSpec((1, H), fix),
            pl.BlockSpec((H, OUT), fix),
            pl.BlockSpec((1, OUT), fix),
        ],
        out_specs=pl.BlockSpec((B, 1), fix),
        out_shape=jax.ShapeDtypeStruct((B, 1), jnp.float32),
        scratch_shapes=[pltpu.VMEM((B, H), jnp.float32)],
    )(h, pq, kindf, w1, b1.reshape(1, H), w2, b2.reshape(1, OUT))


def kernel(protein_pos, protein_atom_feature, ligand_pos, ligand_atom_feature,
           params, batch_protein, batch_ligand, output_kind):
    f32 = jnp.float32
    bp = batch_protein.astype(jnp.int32)
    bl = batch_ligand.astype(jnp.int32)

    ar_b = jnp.arange(B + 1, dtype=jnp.int32)
    pstart = jnp.sum(bp[None, :] < ar_b[:, None], axis=1,
                     dtype=jnp.int32)
    lstart = jnp.sum(bl[None, :] < ar_b[:, None], axis=1, dtype=jnp.int32)
    cnt_p = pstart[1:] - pstart[:-1]
    seg = pstart + lstart
    nidx = jnp.arange(NPAD, dtype=jnp.int32)
    ge = nidx[:, None] >= seg[None, 1:]
    bm = jnp.sum(ge, axis=1, dtype=jnp.int32)
    bmc = jnp.minimum(bm, B - 1)
    oh = (bmc[:, None] == ar_b[None, :B])
    ohi = oh.astype(jnp.int32)
    seg_b = jnp.sum(ohi * seg[None, :B], axis=1)
    pst_b = jnp.sum(ohi * pstart[None, :B], axis=1)
    lst_b = jnp.sum(ohi * lstart[None, :B], axis=1)
    cnt_b = jnp.sum(ohi * cnt_p[None, :], axis=1)
    within = nidx - seg_b
    isprot = within < cnt_b
    orig = jnp.where(isprot, pst_b + within,
                     NP_ + lst_b + within - cnt_b)
    order = jnp.where(nidx < N_, orig, nidx).astype(jnp.int32)

    rb0 = jnp.arange(_NRB, dtype=jnp.int32) * RB
    rb1 = jnp.minimum(rb0 + RB - 1, N_ - 1)
    segq = seg[None, 1:]
    b0 = jnp.minimum(jnp.sum(rb0[:, None] >= segq, axis=1, dtype=jnp.int32),
                     B - 1)
    b1 = jnp.minimum(jnp.sum(rb1[:, None] >= segq, axis=1, dtype=jnp.int32),
                     B - 1)
    ohb = (b0[:, None] == ar_b[None, :B]).astype(jnp.int32)
    wstart = jnp.sum(ohb * seg[None, :B], axis=1)
    ohe = ((b1 + 1)[:, None] == ar_b[None, :]).astype(jnp.int32)
    wend = jnp.maximum(jnp.sum(ohe * seg[None, :], axis=1), wstart + 1)
    jlo = wstart // CT
    ntiles = (wend - 1) // CT - jlo + 1

    hp = _embed(protein_atom_feature, params['protein_emb_W'],
                params['protein_emb_b'], 320)
    hl = _embed(ligand_atom_feature, params['ligand_emb_W'],
                params['ligand_emb_b'], 400)
    h_un = jnp.concatenate(
        [hp, hl, jnp.zeros((NPAD - N_, H), f32)], axis=0)
    pos_un = jnp.concatenate([protein_pos, ligand_pos,
                              jnp.zeros((NPAD - N_, 3), f32)], axis=0)
    batch_un = jnp.concatenate(
        [bp, bl, jnp.full((NPAD - N_,), B, jnp.int32)], axis=0)
    pq_un = jnp.concatenate(
        [pos_un, batch_un[:, None].astype(f32),
         jnp.zeros((NPAD, H - 4), f32)], axis=1)

    h, pq_w = _sc_permute(h_un, pq_un, order)
    pq = pq_w[:, :16]

    d2k, nbr = _knn(pq, jlo, ntiles)

    idxg = nbr.reshape(TOT)
    wnext0 = params['edge_W1_0'][H:2 * H]
    bvec = pl.pallas_call(
        lambda h_ref, w_ref, o_ref: o_ref.__setitem__(
            ..., jnp.dot(h_ref[...], w_ref[...],
                         preferred_element_type=jnp.float32)),
        grid=(_NBD,),
        in_specs=[pl.BlockSpec((BD, H), lambda i: (i, 0)),
                  pl.BlockSpec((H, H), lambda i: (0, 0))],
        out_specs=pl.BlockSpec((BD, H), lambda i: (i, 0)),
        out_shape=jax.ShapeDtypeStruct((NPAD, H), f32),
    )(h, wnext0)

    for i in range(NLAYERS):
        w1 = params['edge_W1_%d' % i]
        w1e = jnp.concatenate([w1[2 * H:], jnp.zeros((NGP - NG, H), f32)],
                              axis=0)
        wn1 = params['node_W1_%d' % i]
        emit_next = i + 1 < NLAYERS
        wnext = (params['edge_W1_%d' % (i + 1)][H:2 * H] if emit_next
                 else jnp.zeros((H, H), f32))
        bs = _sc_gather(bvec, idxg).reshape(K, NPAD, H)
        wd = (w1[:H], w1e, params['edge_b1_%d' % i].reshape(1, H),
              params['edge_W2_%d' % i],
              params['edge_b2_%d' % i].reshape(1, H),
              wn1[:H], wn1[H:], params['node_b1_%d' % i].reshape(1, H),
              params['node_W2_%d' % i],
              params['node_b2_%d' % i].reshape(1, H),
              params['ln_g_%d' % i].reshape(1, H),
              params['ln_b_%d' % i].reshape(1, H),
              wnext)
        h, bvec = _layer(h, bs, d2k, wd, emit_next)

    kindf = output_kind.astype(f32).reshape(B, 1)
    return _readout(h, pq, kindf, params['out_W1'], params['out_b1'],
                    params['out_W2'], params['out_b2'])


# speedup vs baseline: 1.1609x; 1.1609x over previous
import functools

import numpy as np

import jax
import jax.numpy as jnp
from jax import lax
from jax.experimental import pallas as pl
from jax.experimental.pallas import tpu as pltpu
from jax.experimental.pallas import tpu_sc as plsc

NP_ = 8000
NL_ = 2000
N_ = NP_ + NL_
NPAD = 10240
FP = 27
FL = 13
H = 128
B = 64
K = 32
NLAYERS = 3
NG = 20
NGP = 24
CUTOFF = 10.0
OUT = 3

RB = 128
CT = 256
BD = 128
BIGM = 1e10
INF = 3e30

_NTC = NPAD // CT
_NRB = NPAD // RB
_NBD = NPAD // BD
TOT = K * NPAD


def _embed_body(f_ref, w_ref, b_ref, o_ref):
    o_ref[...] = jnp.dot(f_ref[...], w_ref[...],
                         preferred_element_type=jnp.float32) + b_ref[...]


def _embed(feat, w, b, blk):
    n, f = feat.shape
    return pl.pallas_call(
        _embed_body,
        grid=(n // blk,),
        in_specs=[
            pl.BlockSpec((blk, f), lambda i: (i, 0)),
            pl.BlockSpec((f, H), lambda i: (0, 0)),
            pl.BlockSpec((1, H), lambda i: (0, 0)),
        ],
        out_specs=pl.BlockSpec((blk, H), lambda i: (i, 0)),
        out_shape=jax.ShapeDtypeStruct((n, H), jnp.float32),
    )(feat, w, b.reshape(1, H))


def _sc_permute(h_un, pq_un, idx):
    mesh = plsc.VectorSubcoreMesh(core_axis_name="c", subcore_axis_name="s")
    info = plsc.get_sparse_core_info()
    nc = info.num_cores

    @functools.partial(
        pl.kernel, mesh=mesh,
        out_type=[jax.ShapeDtypeStruct((NPAD, H), jnp.float32),
                  jax.ShapeDtypeStruct((NPAD, H), jnp.float32)],
        scratch_types=[
            pltpu.VMEM((320,), jnp.int32),
            pltpu.VMEM((64, H), jnp.float32),
            pltpu.VMEM((64, H), jnp.float32),
            pltpu.SemaphoreType.DMA,
        ],
    )
    def k(h_hbm, pq_hbm, idx_hbm, oh_hbm, opq_hbm, idx_v, hrows, pqrows, sem):
        wid = lax.axis_index("s") * nc + lax.axis_index("c")
        pltpu.sync_copy(idx_hbm.at[pl.ds(wid * 320, 320)], idx_v)

        def body(j, _):
            base = wid * 320 + j * 64
            ids = idx_v.at[pl.ds(j * 64, 64)]
            pltpu.async_copy(h_hbm.at[ids], hrows, sem).wait()
            pltpu.sync_copy(hrows, oh_hbm.at[pl.ds(base, 64)])
            pltpu.async_copy(pq_hbm.at[ids], pqrows, sem).wait()
            pltpu.sync_copy(pqrows, opq_hbm.at[pl.ds(base, 64)])
            return 0

        lax.fori_loop(0, 5, body, 0)

    return k(h_un, pq_un, idx)


def _sc_gather(table, idx):
    mesh = plsc.VectorSubcoreMesh(core_axis_name="c", subcore_axis_name="s")
    info = plsc.get_sparse_core_info()
    nc = info.num_cores

    @functools.partial(
        pl.kernel, mesh=mesh,
        out_type=jax.ShapeDtypeStruct((TOT, H), jnp.float32),
        scratch_types=[
            pltpu.VMEM((10240,), jnp.int32),
            pltpu.VMEM((128, H), jnp.float32),
            pltpu.VMEM((128, H), jnp.float32),
            pltpu.SemaphoreType.DMA,
            pltpu.SemaphoreType.DMA,
        ],
    )
    def k(tbl_hbm, idx_hbm, out_hbm, idx_v, r0, r1, g0, g1):
        wid = lax.axis_index("s") * nc + lax.axis_index("c")
        base = wid * 10240
        pltpu.sync_copy(idx_hbm.at[pl.ds(base, 10240)], idx_v)
        rows = (r0, r1)
        sems = (g0, g1)
        pltpu.async_copy(tbl_hbm.at[idx_v.at[pl.ds(0, 128)]], r0, g0)

        def body(j2, _):
            for b in range(2):
                j = 2 * j2 + b
                nb = 1 - b

                @pl.when(j + 1 < 80)
                def _():
                    ids = idx_v.at[pl.ds((j + 1) * 128, 128)]
                    pltpu.async_copy(tbl_hbm.at[ids], rows[nb], sems[nb])

                pltpu.make_async_copy(
                    tbl_hbm.at[pl.ds(0, 128)], rows[b], sems[b]).wait()
                pltpu.sync_copy(
                    rows[b], out_hbm.at[pl.ds(base + j * 128, 128)])
            return 0

        lax.fori_loop(0, 40, body, 0)

    return k(table, idx)


def _knn_body(jlo_ref, nt_ref, prow_ref, pq_any, outd_ref, outi_ref,
              ptile, sem):
    i = pl.program_id(0)
    pr = prow_ref[...]
    lane16 = lax.broadcasted_iota(jnp.int32, (RB, 16), 1)
    xyz_m = (lane16 < 3).astype(jnp.float32)
    pxyz_r = pr * xyz_m
    sq_r = jnp.sum(pxyz_r * pxyz_r, axis=1, keepdims=True)
    b_r = pr[:, 3:4]
    ar = pxyz_r + jnp.where(lane16 == 3, sq_r, 0.0) \
        + jnp.where(lane16 == 4, 1.0, 0.0)
    rg = (i * RB + lax.broadcasted_iota(jnp.int32, (RB, 1), 0)).astype(
        jnp.float32)

    jlo = jlo_ref[i]
    nt = nt_ref[i]

    pltpu.make_async_copy(pq_any.at[pl.ds(jlo * CT, CT)], ptile.at[0],
                          sem.at[0]).start()

    def tile_step(t, carry):
        bd, bi = carry
        jt = jlo + t
        p = lax.rem(t, 2)
        pltpu.make_async_copy(pq_any.at[pl.ds(jt * CT, CT)], ptile.at[p],
                              sem.at[p]).wait()

        @pl.when(t + 1 < nt)
        def _():
            pltpu.make_async_copy(
                pq_any.at[pl.ds((jt + 1) * CT, CT)],
                ptile.at[lax.rem(t + 1, 2)], sem.at[lax.rem(t + 1, 2)]).start()

        pc = ptile[p]
        lane16c = lax.broadcasted_iota(jnp.int32, (CT, 16), 1)
        pxyz_c = pc * (lane16c < 3).astype(jnp.float32)
        sq_c = jnp.sum(pxyz_c * pxyz_c, axis=1, keepdims=True)
        b_c = pc[:, 3:4]
        ac = -2.0 * pxyz_c + jnp.where(lane16c == 3, 1.0, 0.0) \
            + jnp.where(lane16c == 4, sq_c, 0.0)
        d2 = lax.dot_general(ar, ac, (((1,), (1,)), ((), ())),
                             preferred_element_type=jnp.float32)
        bm = lax.dot_general(
            jnp.concatenate([b_r * b_r, b_r, jnp.ones_like(b_r)], axis=1),
            jnp.concatenate([jnp.ones_like(b_c), -2.0 * b_c, b_c * b_c],
                            axis=1),
            (((1,), (1,)), ((), ())),
            preferred_element_type=jnp.float32)
        cg = (jt * CT
              + lax.broadcasted_iota(jnp.int32, (RB, CT), 1)).astype(
                  jnp.float32)
        d2 = jnp.where(bm != 0.0, BIGM, d2)
        d2 = jnp.where(cg == rg, BIGM, d2)
        lanes = lax.broadcasted_iota(jnp.int32, (RB, K), 1)
        nbd = jnp.full((RB, K), INF, jnp.float32)
        nbi = jnp.zeros((RB, K), jnp.float32)
        for tt in range(K):
            m = jnp.minimum(jnp.min(bd, axis=1, keepdims=True),
                            jnp.min(d2, axis=1, keepdims=True))
            za = jnp.where(bd == m, bi, 3e30)
            zb = jnp.where(d2 == m, cg, 3e30)
            gi = jnp.minimum(jnp.min(za, axis=1, keepdims=True),
                             jnp.min(zb, axis=1, keepdims=True))
            bd = jnp.where(za == gi, INF, bd)
            d2 = jnp.where(zb == gi, INF, d2)
            sel = lanes == tt
            nbd = jnp.where(sel, m, nbd)
            nbi = jnp.where(sel, gi, nbi)
        return (nbd, nbi)

    bd0 = jnp.full((RB, K), INF, jnp.float32)
    bi0 = jnp.zeros((RB, K), jnp.float32)
    bd, bi = lax.fori_loop(0, nt, tile_step, (bd0, bi0))
    outd_ref[...] = bd
    outi_ref[...] = jnp.transpose(bi.astype(jnp.int32))


def _knn(pq, jlo, ntiles):
    grid_spec = pltpu.PrefetchScalarGridSpec(
        num_scalar_prefetch=2,
        grid=(_NRB,),
        in_specs=[
            pl.BlockSpec((RB, 16), lambda i, jlo, nt: (i, 0)),
            pl.BlockSpec(memory_space=pl.ANY),
        ],
        out_specs=[
            pl.BlockSpec((RB, K), lambda i, jlo, nt: (i, 0)),
            pl.BlockSpec((K, RB), lambda i, jlo, nt: (0, i)),
        ],
        scratch_shapes=[
            pltpu.VMEM((2, CT, 16), jnp.float32),
            pltpu.SemaphoreType.DMA((2,)),
        ],
    )
    return pl.pallas_call(
        _knn_body,
        grid_spec=grid_spec,
        out_shape=[jax.ShapeDtypeStruct((NPAD, K), jnp.float32),
                   jax.ShapeDtypeStruct((K, NPAD), jnp.int32)],
    )(jlo, ntiles, pq, pq)


_CSTEP = float(CUTOFF) / (NG - 1)
_W = CUTOFF / NG
_INV2W2 = 1.0 / (2.0 * _W * _W)


def _layer_body(h_ref, bs_ref, d2_ref, w1d_ref, w1e_ref, b1_ref, w2_ref,
                b2_ref, wn1h_ref, wn1a_ref, bn1_ref, wn2_ref, bn2_ref,
                g_ref, be_ref, wnext_ref, oh_ref, ob_ref, *, emit_next):
    h = h_ref[...]
    a = jnp.dot(h, w1d_ref[...], preferred_element_type=jnp.float32) \
        + b1_ref[...]
    cen = lax.broadcasted_iota(jnp.int32, (1, NGP), 1).astype(
        jnp.float32) * _CSTEP
    s0 = jnp.zeros((BD, H), jnp.float32)
    s1 = jnp.zeros((BD, H), jnp.float32)
    for k in range(K):
        dk = d2_ref[:, k:k + 1]
        dist = jnp.sqrt(dk + 1e-12)
        ek = jnp.exp(-((dist - cen) ** 2) * _INV2W2)
        m1 = a + bs_ref[k] + jnp.dot(ek, w1e_ref[...],
                                     preferred_element_type=jnp.float32)
        if k % 2 == 0:
            s0 = s0 + jnp.maximum(m1, 0.0)
        else:
            s1 = s1 + jnp.maximum(m1, 0.0)
    agg = jnp.dot(s0 + s1, w2_ref[...], preferred_element_type=jnp.float32) \
        + float(K) * b2_ref[...]
    u = jnp.maximum(
        jnp.dot(h, wn1h_ref[...], preferred_element_type=jnp.float32)
        + jnp.dot(agg, wn1a_ref[...], preferred_element_type=jnp.float32)
        + bn1_ref[...], 0.0)
    u = jnp.dot(u, wn2_ref[...], preferred_element_type=jnp.float32) \
        + bn2_ref[...]
    r = h + u
    mu = jnp.mean(r, axis=1, keepdims=True)
    var = jnp.mean((r - mu) ** 2, axis=1, keepdims=True)
    hn = (r - mu) * lax.rsqrt(var + 1e-5) * g_ref[...] + be_ref[...]
    oh_ref[...] = hn
    if emit_next:
        ob_ref[...] = jnp.dot(hn, wnext_ref[...],
                              preferred_element_type=jnp.float32)


def _layer(h, bs, d2k, wd, emit_next):
    row = lambda i: (i, 0)
    fix = lambda i: (0, 0)
    in_specs = [
        pl.BlockSpec((BD, H), row),
        pl.BlockSpec((K, BD, H), lambda i: (0, i, 0)),
        pl.BlockSpec((BD, K), row),
        pl.BlockSpec((H, H), fix),
        pl.BlockSpec((NGP, H), fix),
        pl.BlockSpec((1, H), fix),
        pl.BlockSpec((H, H), fix),
        pl.BlockSpec((1, H), fix),
        pl.BlockSpec((H, H), fix),
        pl.BlockSpec((H, H), fix),
        pl.BlockSpec((1, H), fix),
        pl.BlockSpec((H, H), fix),
        pl.BlockSpec((1, H), fix),
        pl.BlockSpec((1, H), fix),
        pl.BlockSpec((1, H), fix),
        pl.BlockSpec((H, H), fix),
    ]
    out_shape = [jax.ShapeDtypeStruct((NPAD, H), jnp.float32),
                 jax.ShapeDtypeStruct((NPAD, H), jnp.float32)]
    out_specs = [pl.BlockSpec((BD, H), row), pl.BlockSpec((BD, H), row)]
    outs = pl.pallas_call(
        functools.partial(_layer_body, emit_next=emit_next),
        grid=(_NBD,),
        in_specs=in_specs,
        out_specs=out_specs,
        out_shape=out_shape,
    )(h, bs, d2k, *wd)
    return outs


def _readout_body(h_ref, pq_ref, kind_ref, w1_ref, b1_ref, w2_ref, b2_ref,
                  out_ref, acc):
    i = pl.program_id(0)

    @pl.when(i == 0)
    def _():
        acc[...] = jnp.zeros((B, H), jnp.float32)

    batch_r = pq_ref[:, 3:4]
    bl = lax.broadcasted_iota(jnp.int32, (RB, B), 1).astype(jnp.float32)
    e = (batch_r == bl).astype(jnp.float32)
    acc[...] += lax.dot_general(e, h_ref[...], (((0,), (0,)), ((), ())),
                                preferred_element_type=jnp.float32)

    @pl.when(i == _NRB - 1)
    def _():
        pre = acc[...]
        x = jnp.dot(pre, w1_ref[...], preferred_element_type=jnp.float32) \
            + b1_ref[...]
        sp = jnp.maximum(x, 0.0) + jnp.log1p(jnp.exp(-jnp.abs(x))) \
            - np.float32(np.log(2.0))
        o3 = jnp.dot(sp, w2_ref[...], preferred_element_type=jnp.float32) \
            + b2_ref[...]
        sel = (lax.broadcasted_iota(jnp.int32, (B, OUT), 1).astype(
            jnp.float32) == kind_ref[...] - 1.0).astype(jnp.float32)
        out_ref[...] = jnp.sum(o3 * sel, axis=1, keepdims=True)


def _readout(h, pq, kindf, w1, b1, w2, b2):
    fix = lambda i: (0, 0)
    return pl.pallas_call(
        _readout_body,
        grid=(_NRB,),
        in_specs=[
            pl.BlockSpec((RB, H), lambda i: (i, 0)),
            pl.BlockSpec((RB, 16), lambda i: (i, 0)),
            pl.BlockSpec((B, 1), fix),
            pl.BlockSpec((H, H), fix),
            pl.BlockSpec((1, H), fix),
            pl.BlockSpec((H, OUT), fix),
            pl.BlockSpec((1, OUT), fix),
        ],
        out_specs=pl.BlockSpec((B, 1), fix),
        out_shape=jax.ShapeDtypeStruct((B, 1), jnp.float32),
        scratch_shapes=[pltpu.VMEM((B, H), jnp.float32)],
    )(h, pq, kindf, w1, b1.reshape(1, H), w2, b2.reshape(1, OUT))


def kernel(protein_pos, protein_atom_feature, ligand_pos, ligand_atom_feature,
           params, batch_protein, batch_ligand, output_kind):
    f32 = jnp.float32
    bp = batch_protein.astype(jnp.int32)
    bl = batch_ligand.astype(jnp.int32)

    ar_b = jnp.arange(B + 1, dtype=jnp.int32)
    pstart = jnp.sum(bp[None, :] < ar_b[:, None], axis=1,
                     dtype=jnp.int32)
    lstart = jnp.sum(bl[None, :] < ar_b[:, None], axis=1, dtype=jnp.int32)
    cnt_p = pstart[1:] - pstart[:-1]
    seg = pstart + lstart
    nidx = jnp.arange(NPAD, dtype=jnp.int32)
    ge = nidx[:, None] >= seg[None, 1:]
    bm = jnp.sum(ge, axis=1, dtype=jnp.int32)
    bmc = jnp.minimum(bm, B - 1)
    oh = (bmc[:, None] == ar_b[None, :B])
    ohi = oh.astype(jnp.int32)
    seg_b = jnp.sum(ohi * seg[None, :B], axis=1)
    pst_b = jnp.sum(ohi * pstart[None, :B], axis=1)
    lst_b = jnp.sum(ohi * lstart[None, :B], axis=1)
    cnt_b = jnp.sum(ohi * cnt_p[None, :], axis=1)
    within = nidx - seg_b
    isprot = within < cnt_b
    orig = jnp.where(isprot, pst_b + within,
                     NP_ + lst_b + within - cnt_b)
    order = jnp.where(nidx < N_, orig, nidx).astype(jnp.int32)

    rb0 = jnp.arange(_NRB, dtype=jnp.int32) * RB
    rb1 = jnp.minimum(rb0 + RB - 1, N_ - 1)
    segq = seg[None, 1:]
    b0 = jnp.minimum(jnp.sum(rb0[:, None] >= segq, axis=1, dtype=jnp.int32),
                     B - 1)
    b1 = jnp.minimum(jnp.sum(rb1[:, None] >= segq, axis=1, dtype=jnp.int32),
                     B - 1)
    ohb = (b0[:, None] == ar_b[None, :B]).astype(jnp.int32)
    wstart = jnp.sum(ohb * seg[None, :B], axis=1)
    ohe = ((b1 + 1)[:, None] == ar_b[None, :]).astype(jnp.int32)
    wend = jnp.maximum(jnp.sum(ohe * seg[None, :], axis=1), wstart + 1)
    jlo = wstart // CT
    ntiles = (wend - 1) // CT - jlo + 1

    hp = _embed(protein_atom_feature, params['protein_emb_W'],
                params['protein_emb_b'], 320)
    hl = _embed(ligand_atom_feature, params['ligand_emb_W'],
                params['ligand_emb_b'], 400)
    h_un = jnp.concatenate(
        [hp, hl, jnp.zeros((NPAD - N_, H), f32)], axis=0)
    pos_un = jnp.concatenate([protein_pos, ligand_pos,
                              jnp.zeros((NPAD - N_, 3), f32)], axis=0)
    batch_un = jnp.concatenate(
        [bp, bl, jnp.full((NPAD - N_,), B, jnp.int32)], axis=0)
    pq_un = jnp.concatenate(
        [pos_un, batch_un[:, None].astype(f32),
         jnp.zeros((NPAD, H - 4), f32)], axis=1)

    h, pq_w = _sc_permute(h_un, pq_un, order)
    pq = pq_w[:, :16]

    d2k, nbr = _knn(pq, jlo, ntiles)

    idxg = nbr.reshape(TOT)
    wnext0 = params['edge_W1_0'][H:2 * H]
    bvec = pl.pallas_call(
        lambda h_ref, w_ref, o_ref: o_ref.__setitem__(
            ..., jnp.dot(h_ref[...], w_ref[...],
                         preferred_element_type=jnp.float32)),
        grid=(_NBD,),
        in_specs=[pl.BlockSpec((BD, H), lambda i: (i, 0)),
                  pl.BlockSpec((H, H), lambda i: (0, 0))],
        out_specs=pl.BlockSpec((BD, H), lambda i: (i, 0)),
        out_shape=jax.ShapeDtypeStruct((NPAD, H), f32),
    )(h, wnext0)

    for i in range(NLAYERS):
        w1 = params['edge_W1_%d' % i]
        w1e = jnp.concatenate([w1[2 * H:], jnp.zeros((NGP - NG, H), f32)],
                              axis=0)
        wn1 = params['node_W1_%d' % i]
        emit_next = i + 1 < NLAYERS
        wnext = (params['edge_W1_%d' % (i + 1)][H:2 * H] if emit_next
                 else jnp.zeros((H, H), f32))
        bs = _sc_gather(bvec, idxg).reshape(K, NPAD, H)
        wd = (w1[:H], w1e, params['edge_b1_%d' % i].reshape(1, H),
              params['edge_W2_%d' % i],
              params['edge_b2_%d' % i].reshape(1, H),
              wn1[:H], wn1[H:], params['node_b1_%d' % i].reshape(1, H),
              params['node_W2_%d' % i],
              params['node_b2_%d' % i].reshape(1, H),
              params['ln_g_%d' % i].reshape(1, H),
              params['ln_b_%d' % i].reshape(1, H),
              wnext)
        h, bvec = _layer(h, bs, d2k, wd, emit_next)

    kindf = output_kind.astype(f32).reshape(B, 1)
    return _readout(h, pq, kindf, params['out_W1'], params['out_b1'],
                    params['out_W2'], params['out_b2'])


# speedup vs baseline: 1.3067x; 1.1255x over previous
import functools

import numpy as np

import jax
import jax.numpy as jnp
from jax import lax
from jax.experimental import pallas as pl
from jax.experimental.pallas import tpu as pltpu
from jax.experimental.pallas import tpu_sc as plsc

NP_ = 8000
NL_ = 2000
N_ = NP_ + NL_
NPAD = 10240
FP = 27
FL = 13
H = 128
B = 64
K = 32
NLAYERS = 3
NG = 20
NGP = 24
CUTOFF = 10.0
OUT = 3

RB = 256
CT = 256
BD = 128
BIGM = 1e10
INF = 3e30

_NTC = NPAD // CT
_NRB = NPAD // RB
_NBD = NPAD // BD
TOT = K * NPAD


def _embed_body(f_ref, w_ref, b_ref, o_ref):
    o_ref[...] = jnp.dot(f_ref[...], w_ref[...],
                         preferred_element_type=jnp.float32) + b_ref[...]


def _embed(feat, w, b, blk):
    n, f = feat.shape
    return pl.pallas_call(
        _embed_body,
        grid=(n // blk,),
        in_specs=[
            pl.BlockSpec((blk, f), lambda i: (i, 0)),
            pl.BlockSpec((f, H), lambda i: (0, 0)),
            pl.BlockSpec((1, H), lambda i: (0, 0)),
        ],
        out_specs=pl.BlockSpec((blk, H), lambda i: (i, 0)),
        out_shape=jax.ShapeDtypeStruct((n, H), jnp.float32),
    )(feat, w, b.reshape(1, H))


def _sc_permute(h_un, pq_un, idx):
    mesh = plsc.VectorSubcoreMesh(core_axis_name="c", subcore_axis_name="s")
    info = plsc.get_sparse_core_info()
    nc = info.num_cores

    @functools.partial(
        pl.kernel, mesh=mesh,
        out_type=[jax.ShapeDtypeStruct((NPAD, H), jnp.float32),
                  jax.ShapeDtypeStruct((NPAD, H), jnp.float32)],
        scratch_types=[
            pltpu.VMEM((320,), jnp.int32),
            pltpu.VMEM((64, H), jnp.float32),
            pltpu.VMEM((64, H), jnp.float32),
            pltpu.SemaphoreType.DMA,
        ],
    )
    def k(h_hbm, pq_hbm, idx_hbm, oh_hbm, opq_hbm, idx_v, hrows, pqrows, sem):
        wid = lax.axis_index("s") * nc + lax.axis_index("c")
        pltpu.sync_copy(idx_hbm.at[pl.ds(wid * 320, 320)], idx_v)

        def body(j, _):
            base = wid * 320 + j * 64
            ids = idx_v.at[pl.ds(j * 64, 64)]
            pltpu.async_copy(h_hbm.at[ids], hrows, sem).wait()
            pltpu.sync_copy(hrows, oh_hbm.at[pl.ds(base, 64)])
            pltpu.async_copy(pq_hbm.at[ids], pqrows, sem).wait()
            pltpu.sync_copy(pqrows, opq_hbm.at[pl.ds(base, 64)])
            return 0

        lax.fori_loop(0, 5, body, 0)

    return k(h_un, pq_un, idx)


def _sc_gather(table, idx):
    mesh = plsc.VectorSubcoreMesh(core_axis_name="c", subcore_axis_name="s")
    info = plsc.get_sparse_core_info()
    nc = info.num_cores

    @functools.partial(
        pl.kernel, mesh=mesh,
        out_type=jax.ShapeDtypeStruct((TOT, H), jnp.float32),
        scratch_types=[
            pltpu.VMEM((10240,), jnp.int32),
            pltpu.VMEM((128, H), jnp.float32),
            pltpu.VMEM((128, H), jnp.float32),
            pltpu.SemaphoreType.DMA,
            pltpu.SemaphoreType.DMA,
        ],
    )
    def k(tbl_hbm, idx_hbm, out_hbm, idx_v, r0, r1, g0, g1):
        wid = lax.axis_index("s") * nc + lax.axis_index("c")
        base = wid * 10240
        pltpu.sync_copy(idx_hbm.at[pl.ds(base, 10240)], idx_v)
        rows = (r0, r1)
        sems = (g0, g1)
        pltpu.async_copy(tbl_hbm.at[idx_v.at[pl.ds(0, 128)]], r0, g0)

        def body(j2, _):
            for b in range(2):
                j = 2 * j2 + b
                nb = 1 - b

                @pl.when(j + 1 < 80)
                def _():
                    ids = idx_v.at[pl.ds((j + 1) * 128, 128)]
                    pltpu.async_copy(tbl_hbm.at[ids], rows[nb], sems[nb])

                pltpu.make_async_copy(
                    tbl_hbm.at[pl.ds(0, 128)], rows[b], sems[b]).wait()
                pltpu.sync_copy(
                    rows[b], out_hbm.at[pl.ds(base + j * 128, 128)])
            return 0

        lax.fori_loop(0, 40, body, 0)

    return k(table, idx)


def _knn_body(jlo_ref, nt_ref, prow_ref, pq_any, outd_ref, outi_ref,
              ptile, sem):
    i = pl.program_id(0)
    pr = prow_ref[...]
    lane16 = lax.broadcasted_iota(jnp.int32, (RB, 16), 1)
    xyz_m = (lane16 < 3).astype(jnp.float32)
    pxyz_r = pr * xyz_m
    sq_r = jnp.sum(pxyz_r * pxyz_r, axis=1, keepdims=True)
    b_r = pr[:, 3:4]
    ar = pxyz_r + jnp.where(lane16 == 3, sq_r, 0.0) \
        + jnp.where(lane16 == 4, 1.0, 0.0)
    rg = (i * RB + lax.broadcasted_iota(jnp.int32, (RB, 1), 0)).astype(
        jnp.float32)

    jlo = jlo_ref[i]
    nt = nt_ref[i]

    def tile_step(t, carry):
        bd, bi = carry
        jt = jlo + t
        cp = pltpu.make_async_copy(pq_any.at[pl.ds(jt * CT, CT)], ptile, sem)
        cp.start()
        cp.wait()
        pc = ptile[...]
        lane16c = lax.broadcasted_iota(jnp.int32, (CT, 16), 1)
        pxyz_c = pc * (lane16c < 3).astype(jnp.float32)
        sq_c = jnp.sum(pxyz_c * pxyz_c, axis=1, keepdims=True)
        b_c = pc[:, 3:4]
        ac = -2.0 * pxyz_c + jnp.where(lane16c == 3, 1.0, 0.0) \
            + jnp.where(lane16c == 4, sq_c, 0.0)
        d2 = lax.dot_general(ar, ac, (((1,), (1,)), ((), ())),
                             preferred_element_type=jnp.float32)
        bm = lax.dot_general(
            jnp.concatenate([b_r * b_r, b_r, jnp.ones_like(b_r)], axis=1),
            jnp.concatenate([jnp.ones_like(b_c), -2.0 * b_c, b_c * b_c],
                            axis=1),
            (((1,), (1,)), ((), ())),
            preferred_element_type=jnp.float32)
        cg = (jt * CT
              + lax.broadcasted_iota(jnp.int32, (RB, CT), 1)).astype(
                  jnp.float32)
        d2 = jnp.where(bm != 0.0, BIGM, d2)
        d2 = jnp.where(cg == rg, BIGM, d2)
        buf = jnp.concatenate([bd, d2], axis=1)
        ibuf = jnp.concatenate([bi, cg], axis=1)
        vals, idxs = [], []
        for _ in range(K):
            m = jnp.min(buf, axis=1, keepdims=True)
            z = jnp.where(buf == m, ibuf, 3e30)
            gi = jnp.min(z, axis=1, keepdims=True)
            buf = jnp.where(z == gi, INF, buf)
            vals.append(m)
            idxs.append(gi)
        return (jnp.concatenate(vals, axis=1), jnp.concatenate(idxs, axis=1))

    bd0 = jnp.full((RB, K), INF, jnp.float32)
    bi0 = jnp.zeros((RB, K), jnp.float32)
    bd, bi = lax.fori_loop(0, nt, tile_step, (bd0, bi0))
    outd_ref[...] = bd
    outi_ref[...] = jnp.transpose(bi.astype(jnp.int32))


def _knn(pq, jlo, ntiles):
    grid_spec = pltpu.PrefetchScalarGridSpec(
        num_scalar_prefetch=2,
        grid=(_NRB,),
        in_specs=[
            pl.BlockSpec((RB, 16), lambda i, jlo, nt: (i, 0)),
            pl.BlockSpec(memory_space=pl.ANY),
        ],
        out_specs=[
            pl.BlockSpec((RB, K), lambda i, jlo, nt: (i, 0)),
            pl.BlockSpec((K, RB), lambda i, jlo, nt: (0, i)),
        ],
        scratch_shapes=[
            pltpu.VMEM((CT, 16), jnp.float32),
            pltpu.SemaphoreType.DMA,
        ],
    )
    return pl.pallas_call(
        _knn_body,
        grid_spec=grid_spec,
        out_shape=[jax.ShapeDtypeStruct((NPAD, K), jnp.float32),
                   jax.ShapeDtypeStruct((K, NPAD), jnp.int32)],
    )(jlo, ntiles, pq, pq)


_CSTEP = float(CUTOFF) / (NG - 1)
_W = CUTOFF / NG
_INV2W2 = 1.0 / (2.0 * _W * _W)


def _layer_body(h_ref, bs_ref, d2_ref, w1d_ref, w1e_ref, b1_ref, w2_ref,
                b2_ref, wn1h_ref, wn1a_ref, bn1_ref, wn2_ref, bn2_ref,
                g_ref, be_ref, wnext_ref, oh_ref, ob_ref, *, emit_next):
    h = h_ref[...]
    a = jnp.dot(h, w1d_ref[...], preferred_element_type=jnp.float32) \
        + b1_ref[...]
    cen = lax.broadcasted_iota(jnp.int32, (1, NGP), 1).astype(
        jnp.float32) * _CSTEP
    s0 = jnp.zeros((BD, H), jnp.float32)
    s1 = jnp.zeros((BD, H), jnp.float32)
    for k in range(K):
        dk = d2_ref[:, k:k + 1]
        dist = jnp.sqrt(dk + 1e-12)
        ek = jnp.exp(-((dist - cen) ** 2) * _INV2W2)
        m1 = a + bs_ref[k] + jnp.dot(ek, w1e_ref[...],
                                     preferred_element_type=jnp.float32)
        if k % 2 == 0:
            s0 = s0 + jnp.maximum(m1, 0.0)
        else:
            s1 = s1 + jnp.maximum(m1, 0.0)
    agg = jnp.dot(s0 + s1, w2_ref[...], preferred_element_type=jnp.float32) \
        + float(K) * b2_ref[...]
    u = jnp.maximum(
        jnp.dot(h, wn1h_ref[...], preferred_element_type=jnp.float32)
        + jnp.dot(agg, wn1a_ref[...], preferred_element_type=jnp.float32)
        + bn1_ref[...], 0.0)
    u = jnp.dot(u, wn2_ref[...], preferred_element_type=jnp.float32) \
        + bn2_ref[...]
    r = h + u
    mu = jnp.mean(r, axis=1, keepdims=True)
    var = jnp.mean((r - mu) ** 2, axis=1, keepdims=True)
    hn = (r - mu) * lax.rsqrt(var + 1e-5) * g_ref[...] + be_ref[...]
    oh_ref[...] = hn
    if emit_next:
        ob_ref[...] = jnp.dot(hn, wnext_ref[...],
                              preferred_element_type=jnp.float32)


def _layer(h, bs, d2k, wd, emit_next):
    row = lambda i: (i, 0)
    fix = lambda i: (0, 0)
    in_specs = [
        pl.BlockSpec((BD, H), row),
        pl.BlockSpec((K, BD, H), lambda i: (0, i, 0)),
        pl.BlockSpec((BD, K), row),
        pl.BlockSpec((H, H), fix),
        pl.BlockSpec((NGP, H), fix),
        pl.BlockSpec((1, H), fix),
        pl.BlockSpec((H, H), fix),
        pl.BlockSpec((1, H), fix),
        pl.BlockSpec((H, H), fix),
        pl.BlockSpec((H, H), fix),
        pl.BlockSpec((1, H), fix),
        pl.BlockSpec((H, H), fix),
        pl.BlockSpec((1, H), fix),
        pl.BlockSpec((1, H), fix),
        pl.BlockSpec((1, H), fix),
        pl.BlockSpec((H, H), fix),
    ]
    out_shape = [jax.ShapeDtypeStruct((NPAD, H), jnp.float32),
                 jax.ShapeDtypeStruct((NPAD, H), jnp.float32)]
    out_specs = [pl.BlockSpec((BD, H), row), pl.BlockSpec((BD, H), row)]
    outs = pl.pallas_call(
        functools.partial(_layer_body, emit_next=emit_next),
        grid=(_NBD,),
        in_specs=in_specs,
        out_specs=out_specs,
        out_shape=out_shape,
    )(h, bs, d2k, *wd)
    return outs


def _readout_body(h_ref, pq_ref, kind_ref, w1_ref, b1_ref, w2_ref, b2_ref,
                  out_ref, acc):
    i = pl.program_id(0)

    @pl.when(i == 0)
    def _():
        acc[...] = jnp.zeros((B, H), jnp.float32)

    batch_r = pq_ref[:, 3:4]
    bl = lax.broadcasted_iota(jnp.int32, (RB, B), 1).astype(jnp.float32)
    e = (batch_r == bl).astype(jnp.float32)
    acc[...] += lax.dot_general(e, h_ref[...], (((0,), (0,)), ((), ())),
                                preferred_element_type=jnp.float32)

    @pl.when(i == _NRB - 1)
    def _():
        pre = acc[...]
        x = jnp.dot(pre, w1_ref[...], preferred_element_type=jnp.float32) \
            + b1_ref[...]
        sp = jnp.maximum(x, 0.0) + jnp.log1p(jnp.exp(-jnp.abs(x))) \
            - np.float32(np.log(2.0))
        o3 = jnp.dot(sp, w2_ref[...], preferred_element_type=jnp.float32) \
            + b2_ref[...]
        sel = (lax.broadcasted_iota(jnp.int32, (B, OUT), 1).astype(
            jnp.float32) == kind_ref[...] - 1.0).astype(jnp.float32)
        out_ref[...] = jnp.sum(o3 * sel, axis=1, keepdims=True)


def _readout(h, pq, kindf, w1, b1, w2, b2):
    fix = lambda i: (0, 0)
    return pl.pallas_call(
        _readout_body,
        grid=(_NRB,),
        in_specs=[
            pl.BlockSpec((RB, H), lambda i: (i, 0)),
            pl.BlockSpec((RB, 16), lambda i: (i, 0)),
            pl.BlockSpec((B, 1), fix),
            pl.BlockSpec((H, H), fix),
            pl.BlockSpec((1, H), fix),
            pl.BlockSpec((H, OUT), fix),
            pl.BlockSpec((1, OUT), fix),
        ],
        out_specs=pl.BlockSpec((B, 1), fix),
        out_shape=jax.ShapeDtypeStruct((B, 1), jnp.float32),
        scratch_shapes=[pltpu.VMEM((B, H), jnp.float32)],
    )(h, pq, kindf, w1, b1.reshape(1, H), w2, b2.reshape(1, OUT))


def kernel(protein_pos, protein_atom_feature, ligand_pos, ligand_atom_feature,
           params, batch_protein, batch_ligand, output_kind):
    f32 = jnp.float32
    bp = batch_protein.astype(jnp.int32)
    bl = batch_ligand.astype(jnp.int32)

    ar_b = jnp.arange(B + 1, dtype=jnp.int32)
    pstart = jnp.sum(bp[None, :] < ar_b[:, None], axis=1,
                     dtype=jnp.int32)
    lstart = jnp.sum(bl[None, :] < ar_b[:, None], axis=1, dtype=jnp.int32)
    cnt_p = pstart[1:] - pstart[:-1]
    seg = pstart + lstart
    nidx = jnp.arange(NPAD, dtype=jnp.int32)
    ge = nidx[:, None] >= seg[None, 1:]
    bm = jnp.sum(ge, axis=1, dtype=jnp.int32)
    bmc = jnp.minimum(bm, B - 1)
    oh = (bmc[:, None] == ar_b[None, :B])
    ohi = oh.astype(jnp.int32)
    seg_b = jnp.sum(ohi * seg[None, :B], axis=1)
    pst_b = jnp.sum(ohi * pstart[None, :B], axis=1)
    lst_b = jnp.sum(ohi * lstart[None, :B], axis=1)
    cnt_b = jnp.sum(ohi * cnt_p[None, :], axis=1)
    within = nidx - seg_b
    isprot = within < cnt_b
    orig = jnp.where(isprot, pst_b + within,
                     NP_ + lst_b + within - cnt_b)
    order = jnp.where(nidx < N_, orig, nidx).astype(jnp.int32)

    rb0 = jnp.arange(_NRB, dtype=jnp.int32) * RB
    rb1 = jnp.minimum(rb0 + RB - 1, N_ - 1)
    segq = seg[None, 1:]
    b0 = jnp.minimum(jnp.sum(rb0[:, None] >= segq, axis=1, dtype=jnp.int32),
                     B - 1)
    b1 = jnp.minimum(jnp.sum(rb1[:, None] >= segq, axis=1, dtype=jnp.int32),
                     B - 1)
    ohb = (b0[:, None] == ar_b[None, :B]).astype(jnp.int32)
    wstart = jnp.sum(ohb * seg[None, :B], axis=1)
    ohe = ((b1 + 1)[:, None] == ar_b[None, :]).astype(jnp.int32)
    wend = jnp.maximum(jnp.sum(ohe * seg[None, :], axis=1), wstart + 1)
    jlo = wstart // CT
    ntiles = (wend - 1) // CT - jlo + 1

    hp = _embed(protein_atom_feature, params['protein_emb_W'],
                params['protein_emb_b'], 320)
    hl = _embed(ligand_atom_feature, params['ligand_emb_W'],
                params['ligand_emb_b'], 400)
    h_un = jnp.concatenate(
        [hp, hl, jnp.zeros((NPAD - N_, H), f32)], axis=0)
    pos_un = jnp.concatenate([protein_pos, ligand_pos,
                              jnp.zeros((NPAD - N_, 3), f32)], axis=0)
    batch_un = jnp.concatenate(
        [bp, bl, jnp.full((NPAD - N_,), B, jnp.int32)], axis=0)
    pq_un = jnp.concatenate(
        [pos_un, batch_un[:, None].astype(f32),
         jnp.zeros((NPAD, H - 4), f32)], axis=1)

    h, pq_w = _sc_permute(h_un, pq_un, order)
    pq = pq_w[:, :16]

    d2k, nbr = _knn(pq, jlo, ntiles)

    idxg = nbr.reshape(TOT)
    wnext0 = params['edge_W1_0'][H:2 * H]
    bvec = pl.pallas_call(
        lambda h_ref, w_ref, o_ref: o_ref.__setitem__(
            ..., jnp.dot(h_ref[...], w_ref[...],
                         preferred_element_type=jnp.float32)),
        grid=(_NBD,),
        in_specs=[pl.BlockSpec((BD, H), lambda i: (i, 0)),
                  pl.BlockSpec((H, H), lambda i: (0, 0))],
        out_specs=pl.BlockSpec((BD, H), lambda i: (i, 0)),
        out_shape=jax.ShapeDtypeStruct((NPAD, H), f32),
    )(h, wnext0)

    for i in range(NLAYERS):
        w1 = params['edge_W1_%d' % i]
        w1e = jnp.concatenate([w1[2 * H:], jnp.zeros((NGP - NG, H), f32)],
                              axis=0)
        wn1 = params['node_W1_%d' % i]
        emit_next = i + 1 < NLAYERS
        wnext = (params['edge_W1_%d' % (i + 1)][H:2 * H] if emit_next
                 else jnp.zeros((H, H), f32))
        bs = _sc_gather(bvec, idxg).reshape(K, NPAD, H)
        wd = (w1[:H], w1e, params['edge_b1_%d' % i].reshape(1, H),
              params['edge_W2_%d' % i],
              params['edge_b2_%d' % i].reshape(1, H),
              wn1[:H], wn1[H:], params['node_b1_%d' % i].reshape(1, H),
              params['node_W2_%d' % i],
              params['node_b2_%d' % i].reshape(1, H),
              params['ln_g_%d' % i].reshape(1, H),
              params['ln_b_%d' % i].reshape(1, H),
              wnext)
        h, bvec = _layer(h, bs, d2k, wd, emit_next)

    kindf = output_kind.astype(f32).reshape(B, 1)
    return _readout(h, pq, kindf, params['out_W1'], params['out_b1'],
                    params['out_W2'], params['out_b2'])
